# baseline (speedup 1.0000x reference)
import functools

import jax
import jax.numpy as jnp
import numpy as np
from jax import lax
from jax.experimental import pallas as pl
from jax.experimental.pallas import tpu as pltpu
from jax.experimental.pallas import tpu_sc as plsc

HID = 128
N0 = 50000
NW = 32
GRP = 16
NGT = N0 // GRP
CHUNK = 1568
NBUF = 4
F32 = jnp.float32


def _table_pass_body(nout, n, blk, *refs):
    h_ref, a_ref, ag_ref = refs[:3]
    c_refs = refs[3:3 + nout]
    vec_ref, ms_ref = refs[3 + nout:]
    i = pl.program_id(0)

    @pl.when(i == 0)
    def _():
        ms_ref[0] = -1e30
        ms_ref[1] = 0.0
        vec_ref[...] = jnp.zeros_like(vec_ref)

    rows = h_ref[...]
    if n % blk:
        gid = i * blk + lax.broadcasted_iota(jnp.int32, (blk, 1), 0)
        rows = jnp.where(gid < n, rows, 0.0)
    for j in range(nout):
        cj = lax.dot_general(
            a_ref[j:j + 1, :], rows, (((1,), (1,)), ((), ())),
            preferred_element_type=F32)
        c_refs[j][...] = cj[0]
    s = lax.dot_general(
        ag_ref[...], rows, (((1,), (1,)), ((), ())),
        preferred_element_type=F32) * (1.0 / np.sqrt(float(HID)))
    if n % blk:
        lid = i * blk + lax.broadcasted_iota(jnp.int32, s.shape, 1)
        s = jnp.where(lid < n, s, -1e30)
    m_old = ms_ref[0]
    m_new = jnp.maximum(m_old, jnp.max(s))
    scale = jnp.exp(m_old - m_new)
    e = jnp.exp(s - m_new)
    ms_ref[1] = ms_ref[1] * scale + jnp.sum(e)
    ms_ref[0] = m_new
    vec_ref[...] = vec_ref[...] * scale + lax.dot_general(
        e, rows, (((1,), (0,)), ((), ())), preferred_element_type=F32)


def _table_pass(h, a, ag):
    n, _ = h.shape
    k = a.shape[0]
    blk = 8192
    grid = -(-n // blk)
    return pl.pallas_call(
        functools.partial(_table_pass_body, k, n, blk),
        grid=(grid,),
        in_specs=[
            pl.BlockSpec((blk, HID), lambda i: (i, 0)),
            pl.BlockSpec((k, HID), lambda i: (0, 0)),
            pl.BlockSpec((1, HID), lambda i: (0, 0)),
        ],
        out_specs=[pl.BlockSpec((blk,), lambda i: (i,))] * k + [
            pl.BlockSpec((1, HID), lambda i: (0, 0)),
            pl.BlockSpec(memory_space=pltpu.SMEM),
        ],
        out_shape=[jax.ShapeDtypeStruct((grid * blk,), F32)] * k + [
            jax.ShapeDtypeStruct((1, HID), F32),
            jax.ShapeDtypeStruct((2,), F32),
        ],
    )(h, a, ag)


def _sc_agg(h_tab, idx, c_vals, d_tab, s_count):
    v = h_tab.shape[0]
    S = s_count
    GS = GRP * S

    mesh = plsc.VectorSubcoreMesh(core_axis_name="c", subcore_axis_name="s")

    @functools.partial(
        pl.kernel,
        mesh=mesh,
        out_type=jax.ShapeDtypeStruct((N0 * HID // 2,), jnp.int32),
        scratch_types=[
            pltpu.VMEM((CHUNK * S,), jnp.int32),
            pltpu.VMEM((CHUNK,), F32),
            pltpu.VMEM((v,), F32),
            pltpu.VMEM((NBUF, GS, HID), F32),
            pltpu.VMEM((NBUF * GRP * HID // 2,), jnp.int32),
            [pltpu.SemaphoreType.DMA] * NBUF,
            [pltpu.SemaphoreType.DMA] * NBUF,
        ],
        compiler_params=pltpu.CompilerParams(needs_layout_passes=False),
    )
    def sc_kernel(h_hbm, idx_hbm, c_hbm, d_hbm, out_hbm,
                  idx_v, c_v, d_v, rows_v, emb_v, sem_g, sem_w):
        wid = lax.axis_index("s") * 2 + lax.axis_index("c")
        gb = (wid * NGT) // NW
        ge = ((wid + 1) * NGT) // NW
        ngl = ge - gb
        nbase = gb * GRP

        pltpu.sync_copy(idx_hbm.at[pl.ds(nbase * S, CHUNK * S)], idx_v)
        pltpu.sync_copy(c_hbm.at[pl.ds(nbase, CHUNK)], c_v)
        pltpu.sync_copy(d_hbm.at[pl.ds(0, v)], d_v)

        lanes = lax.broadcasted_iota(jnp.int32, (16,), 0)

        for b in range(NBUF):
            pltpu.async_copy(
                h_hbm.at[idx_v.at[pl.ds(b * GS, GS)]],
                rows_v.at[b], sem_g[b])

        @pl.loop(0, ngl, step=NBUF)
        def _grp_loop(g0):
            for b in range(NBUF):
                g = g0 + b

                @pl.when(g < ngl)
                def _():
                    pltpu.make_async_copy(
                        h_hbm.at[idx_v.at[pl.ds(0, GS)]],
                        rows_v.at[b], sem_g[b]).wait()

                    cvec = c_v[pl.ds(g * GRP, GRP)]
                    logits = []
                    for s in range(S):
                        pos = g * GS + lanes * S + s
                        nid = plsc.load_gather(idx_v, [pos])
                        gsc = plsc.load_gather(d_v, [nid])
                        x = cvec + gsc
                        logits.append(jnp.where(x >= 0.0, x, 0.01 * x))
                    m = logits[0]
                    for s in range(1, S):
                        m = jnp.maximum(m, logits[s])
                    exps = [jnp.exp(l - m) for l in logits]
                    den = exps[0]
                    for s in range(1, S):
                        den = den + exps[s]
                    rden = 1.0 / den
                    wvecs = [exps[s] * rden for s in range(S)]

                    for i in range(GRP):
                        acc = [jnp.zeros((16,), F32) for _ in range(8)]
                        for s in range(S):
                            wv = jnp.broadcast_to(wvecs[s][i], (16,))
                            r0 = i * S + s
                            for f in range(8):
                                acc[f] = acc[f] + wv * rows_v[b, r0, pl.ds(f * 16, 16)]
                        elu = []
                        for f in range(8):
                            a = acc[f]
                            elu.append(jnp.where(a > 0.0, a, jnp.exp(a) - 1.0))
                        ebase = (b * GRP + i) * (HID // 2)
                        for f in range(4):
                            pk = plsc.pack(
                                elu[2 * f], elu[2 * f + 1],
                                format=plsc.PackFormat.INTERLEAVED)
                            emb_v[pl.ds(ebase + f * 16, 16)] = plsc.bitcast(
                                pk, jnp.int32)

                    @pl.when(g + NBUF < ngl)
                    def _():
                        pltpu.async_copy(
                            h_hbm.at[idx_v.at[pl.ds((g + NBUF) * GS, GS)]],
                            rows_v.at[b], sem_g[b])

                    hw = HID // 2
                    @pl.when(g >= NBUF)
                    def _():
                        pltpu.make_async_copy(
                            emb_v.at[pl.ds(b * GRP * hw, GRP * hw)],
                            out_hbm.at[pl.ds(0, GRP * hw)], sem_w[b]).wait()
                    pltpu.async_copy(
                        emb_v.at[pl.ds(b * GRP * hw, GRP * hw)],
                        out_hbm.at[pl.ds((nbase + g * GRP) * hw, GRP * hw)],
                        sem_w[b])

        for b in range(NBUF):
            pltpu.make_async_copy(
                emb_v.at[pl.ds(b * GRP * HID // 2, GRP * HID // 2)],
                out_hbm.at[pl.ds(0, GRP * HID // 2)], sem_w[b]).wait()

    out32 = sc_kernel(h_tab, idx, c_vals, d_tab)
    return lax.bitcast_convert_type(
        out32, jnp.bfloat16).reshape(N0, HID)


def _beta_pass_body(e_ref, w_ref, b_ref, ai_ref, out_ref):
    i = pl.program_id(0)

    @pl.when(i == 0)
    def _():
        out_ref[0] = 0.0

    blk = e_ref[...]
    sp = jnp.tanh(
        lax.dot_general(blk, w_ref[...], (((1,), (1,)), ((), ())),
                        preferred_element_type=F32) + b_ref[...])
    t = lax.dot_general(sp, ai_ref[...], (((1,), (1,)), ((), ())),
                        preferred_element_type=F32)
    out_ref[0] += jnp.sum(t)


def _beta_pass(e, w, b2, ai):
    blk = 5000
    grid = N0 // blk
    return pl.pallas_call(
        _beta_pass_body,
        grid=(grid,),
        in_specs=[
            pl.BlockSpec((blk, HID), lambda i: (i, 0)),
            pl.BlockSpec((HID, HID), lambda i: (0, 0)),
            pl.BlockSpec((1, HID), lambda i: (0, 0)),
            pl.BlockSpec((1, HID), lambda i: (0, 0)),
        ],
        out_specs=pl.BlockSpec(memory_space=pltpu.SMEM),
        out_shape=jax.ShapeDtypeStruct((1,), F32),
    )(e, w, b2, ai)


def _fuse_body(e0_ref, e1_ref, pm_ref, v0_ref, v1_ref, v2_ref,
               ms_ref, bs_ref, gate_ref, out_ref):
    m = jnp.maximum(jnp.maximum(ms_ref[0], ms_ref[2]), ms_ref[4])
    s0 = jnp.exp(ms_ref[0] - m)
    s1 = jnp.exp(ms_ref[2] - m)
    s2 = jnp.exp(ms_ref[4] - m)
    den = ms_ref[1] * s0 + ms_ref[3] * s1 + ms_ref[5] * s2
    gvec = (v0_ref[...] * s0 + v1_ref[...] * s1 + v2_ref[...] * s2) / den

    b0 = bs_ref[0] / float(N0)
    b1 = bs_ref[1] / float(N0)
    bm = jnp.maximum(b0, b1)
    x0 = jnp.exp(b0 - bm)
    x1 = jnp.exp(b1 - bm)
    beta0 = x0 / (x0 + x1)
    beta1 = x1 / (x0 + x1)

    gv = 1.0 / (1.0 + jnp.exp(-gate_ref[0]))
    z = gv * (beta0 * e0_ref[...].astype(F32)
              + beta1 * e1_ref[...].astype(F32))
    zt = lax.dot_general(z, pm_ref[...], (((1,), (0,)), ((), ())),
                         preferred_element_type=F32)
    out_ref[...] = zt + (1.0 - gv) * gvec


def _fuse(e0, e1, pm, v0, v1, v2, ms, bs, gate1):
    blk = 5000
    grid = N0 // blk
    return pl.pallas_call(
        _fuse_body,
        grid=(grid,),
        in_specs=[
            pl.BlockSpec((blk, HID), lambda i: (i, 0)),
            pl.BlockSpec((blk, HID), lambda i: (i, 0)),
            pl.BlockSpec((HID, HID), lambda i: (0, 0)),
            pl.BlockSpec((1, HID), lambda i: (0, 0)),
            pl.BlockSpec((1, HID), lambda i: (0, 0)),
            pl.BlockSpec((1, HID), lambda i: (0, 0)),
            pl.BlockSpec(memory_space=pltpu.SMEM),
            pl.BlockSpec(memory_space=pltpu.SMEM),
            pl.BlockSpec(memory_space=pltpu.SMEM),
        ],
        out_specs=pl.BlockSpec((blk, HID), lambda i: (i, 0)),
        out_shape=jax.ShapeDtypeStruct((N0, HID), F32),
    )(e0, e1, pm, v0, v1, v2, ms, bs, gate1)


def kernel(nei_h_0, nei_h_1, nei_h_2, nei_idx_0, nei_idx_1,
           att_intra0, att_intra1, fc_W, fc_b, att_inter, att_global, gate):
    a0r = att_intra0[:, :HID]
    a0n = att_intra0[:, HID:]
    a1r = att_intra1[:, :HID]
    a1n = att_intra1[:, HID:]
    arefs = jnp.concatenate([a0r, a1r], axis=0)

    c0, c1, vec0, ms0 = _table_pass(nei_h_0, arefs, att_global)
    d0, vec1, ms1 = _table_pass(nei_h_1, a0n, att_global)
    d1, vec2, ms2 = _table_pass(nei_h_2, a1n, att_global)

    idx0f = nei_idx_0.astype(jnp.int32).reshape(-1)
    idx1f = nei_idx_1.astype(jnp.int32).reshape(-1)
    emb0 = _sc_agg(nei_h_1, idx0f, c0, d0, 4)
    emb1 = _sc_agg(nei_h_2, idx1f, c1, d1, 8)

    perm = np.zeros(HID, np.int32)
    for cch in range(4):
        for j in range(16):
            for t in range(2):
                perm[32 * cch + 2 * j + t] = 32 * cch + 16 * t + j
    fc_Wp = fc_W[:, jnp.asarray(perm)]
    pmat = np.zeros((HID, HID), np.float32)
    pmat[np.arange(HID), perm] = 1.0
    pmat = jnp.asarray(pmat)
    b2 = fc_b.reshape(1, HID)
    bs0 = _beta_pass(emb0, fc_Wp, b2, att_inter)
    bs1 = _beta_pass(emb1, fc_Wp, b2, att_inter)

    ms = jnp.concatenate([ms0, ms1, ms2])
    bs = jnp.concatenate([bs0, bs1])
    gate1 = jnp.reshape(gate, (1,)).astype(F32)

    return _fuse(emb0, emb1, pmat, vec0, vec1, vec2, ms, bs, gate1)


# speedup vs baseline: 1.5937x; 1.5937x over previous
import functools

import jax
import jax.numpy as jnp
import numpy as np
from jax import lax
from jax.experimental import pallas as pl
from jax.experimental.pallas import tpu as pltpu
from jax.experimental.pallas import tpu_sc as plsc

HID = 128
N0 = 50000
NW = 32
GRP = 16
NGT = N0 // GRP
CHUNK = 1568
NBUF = 4
F32 = jnp.float32


def _table_pass_body(nout, n, blk, *refs):
    h_ref, a_ref, ag_ref = refs[:3]
    c_refs = refs[3:3 + nout]
    vec_ref, ms_ref = refs[3 + nout:]
    i = pl.program_id(0)

    @pl.when(i == 0)
    def _():
        ms_ref[0] = -1e30
        ms_ref[1] = 0.0
        vec_ref[...] = jnp.zeros_like(vec_ref)

    rows = h_ref[...]
    if n % blk:
        gid = i * blk + lax.broadcasted_iota(jnp.int32, (blk, 1), 0)
        rows = jnp.where(gid < n, rows, 0.0)
    for j in range(nout):
        cj = lax.dot_general(
            a_ref[j:j + 1, :], rows, (((1,), (1,)), ((), ())),
            preferred_element_type=F32)
        c_refs[j][...] = cj[0]
    s = lax.dot_general(
        ag_ref[...], rows, (((1,), (1,)), ((), ())),
        preferred_element_type=F32) * (1.0 / np.sqrt(float(HID)))
    if n % blk:
        lid = i * blk + lax.broadcasted_iota(jnp.int32, s.shape, 1)
        s = jnp.where(lid < n, s, -1e30)
    m_old = ms_ref[0]
    m_new = jnp.maximum(m_old, jnp.max(s))
    scale = jnp.exp(m_old - m_new)
    e = jnp.exp(s - m_new)
    ms_ref[1] = ms_ref[1] * scale + jnp.sum(e)
    ms_ref[0] = m_new
    vec_ref[...] = vec_ref[...] * scale + lax.dot_general(
        e, rows, (((1,), (0,)), ((), ())), preferred_element_type=F32)


def _table_pass(h, a, ag):
    n, _ = h.shape
    k = a.shape[0]
    blk = 8192
    grid = -(-n // blk)
    return pl.pallas_call(
        functools.partial(_table_pass_body, k, n, blk),
        grid=(grid,),
        in_specs=[
            pl.BlockSpec((blk, HID), lambda i: (i, 0)),
            pl.BlockSpec((k, HID), lambda i: (0, 0)),
            pl.BlockSpec((1, HID), lambda i: (0, 0)),
        ],
        out_specs=[pl.BlockSpec((blk,), lambda i: (i,))] * k + [
            pl.BlockSpec((1, HID), lambda i: (0, 0)),
            pl.BlockSpec(memory_space=pltpu.SMEM),
        ],
        out_shape=[jax.ShapeDtypeStruct((grid * blk,), F32)] * k + [
            jax.ShapeDtypeStruct((1, HID), F32),
            jax.ShapeDtypeStruct((2,), F32),
        ],
    )(h, a, ag)


def _sc_agg(h_tab, idx, c_vals, d_tab, s_count):
    v = h_tab.shape[0]
    S = s_count
    GS = GRP * S

    mesh = plsc.VectorSubcoreMesh(core_axis_name="c", subcore_axis_name="s")

    @functools.partial(
        pl.kernel,
        mesh=mesh,
        out_type=jax.ShapeDtypeStruct((N0 // 2, HID), jnp.int32),
        scratch_types=[
            pltpu.VMEM((CHUNK * S,), jnp.int32),
            pltpu.VMEM((CHUNK,), F32),
            pltpu.VMEM((v,), F32),
            pltpu.VMEM((NBUF, GS, HID), F32),
            pltpu.VMEM((NBUF, GRP // 2, HID), jnp.int32),
            [pltpu.SemaphoreType.DMA] * NBUF,
            [pltpu.SemaphoreType.DMA] * NBUF,
        ],
        compiler_params=pltpu.CompilerParams(needs_layout_passes=False),
    )
    def sc_kernel(h_hbm, idx_hbm, c_hbm, d_hbm, out_hbm,
                  idx_v, c_v, d_v, rows_v, emb_v, sem_g, sem_w):
        wid = lax.axis_index("s") * 2 + lax.axis_index("c")
        gb = (wid * NGT) // NW
        ge = ((wid + 1) * NGT) // NW
        ngl = ge - gb
        nbase = gb * GRP

        pltpu.sync_copy(idx_hbm.at[pl.ds(nbase * S, CHUNK * S)], idx_v)
        pltpu.sync_copy(c_hbm.at[pl.ds(nbase, CHUNK)], c_v)
        pltpu.sync_copy(d_hbm.at[pl.ds(0, v)], d_v)

        lanes = lax.broadcasted_iota(jnp.int32, (16,), 0)

        for b in range(NBUF):
            pltpu.async_copy(
                h_hbm.at[idx_v.at[pl.ds(b * GS, GS)]],
                rows_v.at[b], sem_g[b])

        @pl.loop(0, ngl, step=NBUF)
        def _grp_loop(g0):
            for b in range(NBUF):
                g = g0 + b

                @pl.when(g < ngl)
                def _():
                    pltpu.make_async_copy(
                        h_hbm.at[idx_v.at[pl.ds(0, GS)]],
                        rows_v.at[b], sem_g[b]).wait()

                    cvec = c_v[pl.ds(g * GRP, GRP)]
                    logits = []
                    for s in range(S):
                        pos = g * GS + lanes * S + s
                        nid = plsc.load_gather(idx_v, [pos])
                        gsc = plsc.load_gather(d_v, [nid])
                        x = cvec + gsc
                        logits.append(jnp.where(x >= 0.0, x, 0.01 * x))
                    m = logits[0]
                    for s in range(1, S):
                        m = jnp.maximum(m, logits[s])
                    exps = [jnp.exp(l - m) for l in logits]
                    den = exps[0]
                    for s in range(1, S):
                        den = den + exps[s]
                    rden = 1.0 / den
                    wvecs = [exps[s] * rden for s in range(S)]

                    for i2 in range(GRP // 2):
                        pair = []
                        for t in range(2):
                            i = 2 * i2 + t
                            acc = [jnp.zeros((16,), F32) for _ in range(8)]
                            for s in range(S):
                                wv = jnp.broadcast_to(wvecs[s][i], (16,))
                                r0 = i * S + s
                                for f in range(8):
                                    acc[f] = acc[f] + wv * rows_v[b, r0, pl.ds(f * 16, 16)]
                            pair.append([
                                jnp.where(a > 0.0, a, jnp.exp(a) - 1.0)
                                for a in acc])
                        for f in range(8):
                            pk = plsc.pack(
                                pair[0][f], pair[1][f],
                                format=plsc.PackFormat.INTERLEAVED)
                            emb_v[b, i2, pl.ds(f * 16, 16)] = plsc.bitcast(
                                pk, jnp.int32)

                    @pl.when(g + NBUF < ngl)
                    def _():
                        pltpu.async_copy(
                            h_hbm.at[idx_v.at[pl.ds((g + NBUF) * GS, GS)]],
                            rows_v.at[b], sem_g[b])

                    @pl.when(g >= NBUF)
                    def _():
                        pltpu.make_async_copy(
                            emb_v.at[b],
                            out_hbm.at[pl.ds(0, GRP // 2)], sem_w[b]).wait()
                    prow = pl.multiple_of((nbase + g * GRP) // 2, 8)
                    pltpu.async_copy(
                        emb_v.at[b],
                        out_hbm.at[pl.ds(prow, GRP // 2)],
                        sem_w[b])

        for b in range(NBUF):
            pltpu.make_async_copy(
                emb_v.at[b], out_hbm.at[pl.ds(0, GRP // 2)], sem_w[b]).wait()

    return sc_kernel(h_tab, idx, c_vals, d_tab)


def _beta_pass_body(e_ref, w_ref, b_ref, ai_ref, out_ref):
    i = pl.program_id(0)

    @pl.when(i == 0)
    def _():
        out_ref[0] = 0.0

    blk = pltpu.bitcast(e_ref[...], jnp.bfloat16).astype(F32)
    sp = jnp.tanh(
        lax.dot_general(blk, w_ref[...], (((1,), (1,)), ((), ())),
                        preferred_element_type=F32) + b_ref[...])
    t = lax.dot_general(sp, ai_ref[...], (((1,), (1,)), ((), ())),
                        preferred_element_type=F32)
    out_ref[0] += jnp.sum(t)


def _beta_pass(e, w, b2, ai):
    blk = 10000
    grid = N0 // blk
    return pl.pallas_call(
        _beta_pass_body,
        grid=(grid,),
        in_specs=[
            pl.BlockSpec((blk // 2, HID), lambda i: (i, 0)),
            pl.BlockSpec((HID, HID), lambda i: (0, 0)),
            pl.BlockSpec((1, HID), lambda i: (0, 0)),
            pl.BlockSpec((1, HID), lambda i: (0, 0)),
        ],
        out_specs=pl.BlockSpec(memory_space=pltpu.SMEM),
        out_shape=jax.ShapeDtypeStruct((1,), F32),
    )(e, w, b2, ai)


def _fuse_body(e0_ref, e1_ref, v0_ref, v1_ref, v2_ref,
               ms_ref, bs_ref, gate_ref, out_ref):
    m = jnp.maximum(jnp.maximum(ms_ref[0], ms_ref[2]), ms_ref[4])
    s0 = jnp.exp(ms_ref[0] - m)
    s1 = jnp.exp(ms_ref[2] - m)
    s2 = jnp.exp(ms_ref[4] - m)
    den = ms_ref[1] * s0 + ms_ref[3] * s1 + ms_ref[5] * s2
    gvec = (v0_ref[...] * s0 + v1_ref[...] * s1 + v2_ref[...] * s2) / den

    b0 = bs_ref[0] / float(N0)
    b1 = bs_ref[1] / float(N0)
    bm = jnp.maximum(b0, b1)
    x0 = jnp.exp(b0 - bm)
    x1 = jnp.exp(b1 - bm)
    beta0 = x0 / (x0 + x1)
    beta1 = x1 / (x0 + x1)

    gv = 1.0 / (1.0 + jnp.exp(-gate_ref[0]))
    e0 = pltpu.bitcast(e0_ref[...], jnp.bfloat16).astype(F32)
    e1 = pltpu.bitcast(e1_ref[...], jnp.bfloat16).astype(F32)
    out_ref[...] = gv * (beta0 * e0 + beta1 * e1) + (1.0 - gv) * gvec


def _fuse(e0, e1, v0, v1, v2, ms, bs, gate1):
    blk = 10000
    grid = N0 // blk
    return pl.pallas_call(
        _fuse_body,
        grid=(grid,),
        in_specs=[
            pl.BlockSpec((blk // 2, HID), lambda i: (i, 0)),
            pl.BlockSpec((blk // 2, HID), lambda i: (i, 0)),
            pl.BlockSpec((1, HID), lambda i: (0, 0)),
            pl.BlockSpec((1, HID), lambda i: (0, 0)),
            pl.BlockSpec((1, HID), lambda i: (0, 0)),
            pl.BlockSpec(memory_space=pltpu.SMEM),
            pl.BlockSpec(memory_space=pltpu.SMEM),
            pl.BlockSpec(memory_space=pltpu.SMEM),
        ],
        out_specs=pl.BlockSpec((blk, HID), lambda i: (i, 0)),
        out_shape=jax.ShapeDtypeStruct((N0, HID), F32),
    )(e0, e1, v0, v1, v2, ms, bs, gate1)


def kernel(nei_h_0, nei_h_1, nei_h_2, nei_idx_0, nei_idx_1,
           att_intra0, att_intra1, fc_W, fc_b, att_inter, att_global, gate):
    a0r = att_intra0[:, :HID]
    a0n = att_intra0[:, HID:]
    a1r = att_intra1[:, :HID]
    a1n = att_intra1[:, HID:]
    arefs = jnp.concatenate([a0r, a1r], axis=0)

    c0, c1, vec0, ms0 = _table_pass(nei_h_0, arefs, att_global)
    d0, vec1, ms1 = _table_pass(nei_h_1, a0n, att_global)
    d1, vec2, ms2 = _table_pass(nei_h_2, a1n, att_global)

    idx0f = nei_idx_0.astype(jnp.int32).reshape(-1)
    idx1f = nei_idx_1.astype(jnp.int32).reshape(-1)
    emb0 = _sc_agg(nei_h_1, idx0f, c0, d0, 4)
    emb1 = _sc_agg(nei_h_2, idx1f, c1, d1, 8)

    b2 = fc_b.reshape(1, HID)
    bs0 = _beta_pass(emb0, fc_W, b2, att_inter)
    bs1 = _beta_pass(emb1, fc_W, b2, att_inter)

    ms = jnp.concatenate([ms0, ms1, ms2])
    bs = jnp.concatenate([bs0, bs1])
    gate1 = jnp.reshape(gate, (1,)).astype(F32)

    return _fuse(emb0, emb1, vec0, vec1, vec2, ms, bs, gate1)
